# Initial kernel scaffold; baseline (speedup 1.0000x reference)
import jax
import jax.numpy as jnp
from jax.experimental import pallas as pl


def kernel(node_features_t, edge_index, edge_features_t, event_type_id, event_params, event_node_mask, event_edge_mask, event_scope_node_mask, event_scope_edge_mask, node_batch_index, edge_batch_index, num_nodes_per_graph, num_edges_per_graph, event_emb, ne_w1, ne_b1, ne_w2, ne_b2, ee_w1, ee_b1, ee_w2, ee_b2, msg_w, msg_b, nu_w, nu_b, nd_w, nd_b, ed_w1, ed_b1, ed_w2, ed_b2):
    raise NotImplementedError("write your pallas kernel here")


# speedup vs baseline: 2.3751x; 2.3751x over previous
import functools

import jax
import jax.numpy as jnp
from jax import lax
from jax.experimental import pallas as pl
from jax.experimental.pallas import tpu as pltpu
from jax.experimental.pallas import tpu_sc as plsc

N = 50000
E = 800000
H = 64
NB = 1000
EB = 4000

_f32 = jnp.float32


def _tc_node_mlp(nf_ref, nbi_ref, ntg_ref, nsc_ref, etid_ref, eemb_ref,
                 eprm_ref, w1_ref, b1_ref, w2_ref, b2_ref, out_ref):
    nf = nf_ref[...]
    bidx = nbi_ref[...]
    tgt = ntg_ref[...]
    sc = nsc_ref[...]
    w1 = w1_ref[...]
    oh2 = (etid_ref[...] == lax.broadcasted_iota(jnp.int32, (1, 2), 1).astype(_f32)).astype(_f32)
    ge = oh2 @ eemb_ref[...]
    tw = ge @ w1[7:23] + eprm_ref[...] @ w1[23:27]
    oh8 = (bidx == lax.broadcasted_iota(jnp.int32, (1, 8), 1).astype(_f32)).astype(_f32)
    c1 = (nf @ w1[0:7] + oh8 @ tw + tgt * w1[27:28] + sc * w1[28:29]
          + b1_ref[...])
    h1 = jnp.maximum(c1, 0.0)
    out_ref[...] = jnp.maximum(h1 @ w2_ref[...] + b2_ref[...], 0.0)


def _tc_edge_mlp(ef_ref, hs_ref, hd_ref, ebi_ref, etg_ref, esc_ref, etid_ref,
                 eemb_ref, eprm_ref, w1_ref, b1_ref, w2_ref, b2_ref, mw_ref,
                 mb_ref, eh_ref, msg_ref):
    ef = ef_ref[...]
    hs = hs_ref[...]
    hd = hd_ref[...]
    bidx = ebi_ref[...]
    tgt = etg_ref[...]
    sc = esc_ref[...]
    w1 = w1_ref[...]
    oh2 = (etid_ref[...] == lax.broadcasted_iota(jnp.int32, (1, 2), 1).astype(_f32)).astype(_f32)
    ge = oh2 @ eemb_ref[...]
    tw = ge @ w1[132:148] + eprm_ref[...] @ w1[148:152]
    oh8 = (bidx == lax.broadcasted_iota(jnp.int32, (1, 8), 1).astype(_f32)).astype(_f32)
    c1 = (ef @ w1[0:4] + hs @ w1[4:68] + hd @ w1[68:132] + oh8 @ tw
          + tgt * w1[152:153] + sc * w1[153:154] + b1_ref[...])
    e1 = jnp.maximum(c1, 0.0)
    ehv = jnp.maximum(e1 @ w2_ref[...] + b2_ref[...], 0.0)
    eh_ref[...] = ehv
    msg_ref[...] = (ehv @ mw_ref[...] + mb_ref[...]) * sc


def _tc_node_update(nh_ref, ag_ref, nf_ref, nsc_ref, nuw_ref, nub_ref,
                    ndw_ref, ndb_ref, nhu_ref, ndp_ref, nfp_ref):
    nh = nh_ref[...]
    ag = ag_ref[...]
    nuw = nuw_ref[...]
    nhu = jnp.maximum(nh @ nuw[0:64] + ag @ nuw[64:128] + nub_ref[...], 0.0)
    raw = nhu @ ndw_ref[...] + ndb_ref[...]
    sc = nsc_ref[...]
    nf = nf_ref[...]
    ndp = raw * sc
    nhu_ref[...] = nhu
    ndp_ref[...] = ndp
    nfp_ref[...] = jnp.where(sc > 0.5, nf + ndp, nf)


def _tc_edge_delta(eh_ref, us_ref, ud_ref, ef_ref, esc_ref, w1_ref, b1_ref,
                   w2_ref, b2_ref, edp_ref, efp_ref):
    w1 = w1_ref[...]
    t = jnp.maximum(eh_ref[...] @ w1[0:64] + us_ref[...] @ w1[64:128]
                    + ud_ref[...] @ w1[128:192] + b1_ref[...], 0.0)
    raw = t @ w2_ref[...] + b2_ref[...]
    sc = esc_ref[...]
    ef = ef_ref[...]
    edp = raw * sc
    edp_ref[...] = edp
    efp_ref[...] = jnp.where(sc > 0.5, ef + edp, ef)


_SC_NW = 32
_GPW = (2 * E) // _SC_NW
_GC = 400
_GI = _GPW // _GC


def _sc_gather_body(table_hbm, idx_hbm, out_hbm, idx_v, rows_v, sem):
    wid = lax.axis_index("s") * 2 + lax.axis_index("c")
    base = wid * _GPW

    def step(i, carry):
        off = base + i * _GC
        pltpu.sync_copy(idx_hbm.at[pl.ds(off, _GC)], idx_v)
        pltpu.async_copy(table_hbm.at[idx_v], rows_v, sem).wait()
        pltpu.sync_copy(rows_v, out_hbm.at[pl.ds(off, _GC)])
        return carry

    lax.fori_loop(0, _GI, step, 0)


_NHALF = N // 2
_ACC_PAD = 25008
_EPT = E // 16
_SCC = 400
_SCI = _EPT // _SCC
_WB = 250


def _sc_scatter_body(msg_hbm, idx_hbm, out_hbm, acc, mbuf, sib, dib, slb, dlb):
    c = lax.axis_index("c")
    s = lax.axis_index("s")
    nbase = c * _NHALF

    zero16 = jnp.zeros((16,), _f32)

    def zstep(r, carry):
        for k in range(4):
            mbuf[r, pl.ds(k * 16, 16)] = zero16
        return carry

    lax.fori_loop(0, _SCC, zstep, 0)
    for q in range(3):
        pltpu.sync_copy(mbuf, acc.at[pl.ds(s * 1563 + q * 400, 400)])
    pltpu.sync_copy(mbuf.at[pl.ds(0, 363)],
                    acc.at[pl.ds(s * 1563 + 1200, 363)])
    plsc.subcore_barrier()

    def step(i, carry):
        ebase = s * _EPT + i * _SCC
        pltpu.sync_copy(msg_hbm.at[pl.ds(ebase, _SCC)], mbuf)
        pltpu.sync_copy(idx_hbm.at[pl.ds(ebase, _SCC)], sib)
        pltpu.sync_copy(idx_hbm.at[pl.ds(E + ebase, _SCC)], dib)
        for j in range(_SCC // 16):
            sv = sib[pl.ds(j * 16, 16)] - nbase
            sv = jnp.where((sv >= 0) & (sv < _NHALF), sv, _NHALF)
            slb[j // 5, pl.ds((j % 5) * 16, 16)] = sv
            dv = dib[pl.ds(j * 16, 16)] - nbase
            dv = jnp.where((dv >= 0) & (dv < _NHALF), dv, _NHALF)
            dlb[j // 5, pl.ds((j % 5) * 16, 16)] = dv
        for q in range(5):
            pltpu.sync_copy(mbuf.at[pl.ds(q * 80, 80)], acc.at[slb.at[q]],
                            add=True)
            pltpu.sync_copy(mbuf.at[pl.ds(q * 80, 80)], acc.at[dlb.at[q]],
                            add=True)
        return carry

    lax.fori_loop(0, _SCI, step, 0)
    plsc.subcore_barrier()

    for k in range(6):
        kid = s + 16 * k
        pltpu.sync_copy(acc.at[pl.ds(kid * _WB, _WB)], mbuf.at[pl.ds(0, _WB)])
        pltpu.sync_copy(mbuf.at[pl.ds(0, _WB)],
                        out_hbm.at[pl.ds(nbase + kid * _WB, _WB)])

    @pl.when(s < 4)
    def _():
        kid = s + 96
        pltpu.sync_copy(acc.at[pl.ds(kid * _WB, _WB)], mbuf.at[pl.ds(0, _WB)])
        pltpu.sync_copy(mbuf.at[pl.ds(0, _WB)],
                        out_hbm.at[pl.ds(nbase + kid * _WB, _WB)])


@functools.cache
def _sc_kernels():
    mesh = plsc.VectorSubcoreMesh(core_axis_name="c", subcore_axis_name="s")
    sc_params = pltpu.CompilerParams(use_tc_tiling_on_sc=False)
    gather = functools.partial(
        pl.kernel,
        out_type=jax.ShapeDtypeStruct((2 * E, H), _f32),
        mesh=mesh,
        compiler_params=sc_params,
        scratch_types=[
            pltpu.VMEM((_GC,), jnp.int32),
            pltpu.VMEM((_GC, H), _f32),
            pltpu.SemaphoreType.DMA,
        ],
    )(_sc_gather_body)
    scatter = functools.partial(
        pl.kernel,
        out_type=jax.ShapeDtypeStruct((N, H), _f32),
        mesh=mesh,
        compiler_params=sc_params,
        scratch_types=[
            pltpu.VMEM_SHARED((_ACC_PAD, H), _f32),
            pltpu.VMEM((_SCC, H), _f32),
            pltpu.VMEM((_SCC,), jnp.int32),
            pltpu.VMEM((_SCC,), jnp.int32),
            pltpu.VMEM((5, 80), jnp.int32),
            pltpu.VMEM((5, 80), jnp.int32),
        ],
    )(_sc_scatter_body)
    return gather, scatter


def _col(x):
    return x.astype(_f32)[:, None]


def kernel(node_features_t, edge_index, edge_features_t, event_type_id,
           event_params, event_node_mask, event_edge_mask,
           event_scope_node_mask, event_scope_edge_mask, node_batch_index,
           edge_batch_index, num_nodes_per_graph, num_edges_per_graph,
           event_emb, ne_w1, ne_b1, ne_w2, ne_b2, ee_w1, ee_b1, ee_w2, ee_b2,
           msg_w, msg_b, nu_w, nu_b, nd_w, nd_b, ed_w1, ed_b1, ed_w2, ed_b2):
    nbi = _col(node_batch_index)
    ntg = _col(event_node_mask)
    nsc = _col(event_scope_node_mask)
    ebi = _col(edge_batch_index)
    etg = _col(event_edge_mask)
    esc = _col(event_scope_edge_mask)
    etid = _col(event_type_id)
    b2d = lambda b: b[None, :]

    full = lambda shape: pl.BlockSpec(shape, lambda i: (0, 0))
    nrow = lambda w: pl.BlockSpec((NB, w), lambda i: (i, 0))
    erow = lambda w: pl.BlockSpec((EB, w), lambda i: (i, 0))
    tc_params = pltpu.CompilerParams(dimension_semantics=("parallel",))

    node_h = pl.pallas_call(
        _tc_node_mlp,
        grid=(N // NB,),
        in_specs=[nrow(7), nrow(1), nrow(1), nrow(1), full((8, 1)),
                  full((2, 16)), full((8, 4)), full((29, H)), full((1, H)),
                  full((H, H)), full((1, H))],
        out_specs=nrow(H),
        out_shape=jax.ShapeDtypeStruct((N, H), _f32),
        compiler_params=tc_params,
    )(node_features_t, nbi, ntg, nsc, etid, event_emb, event_params,
      ne_w1, b2d(ne_b1), ne_w2, b2d(ne_b2))

    sc_gather, sc_scatter = _sc_kernels()
    flat_idx = edge_index.reshape(2 * E)
    hpair = sc_gather(node_h, flat_idx)

    src_spec = pl.BlockSpec((EB, H), lambda i: (i, 0))
    dst_spec = pl.BlockSpec((EB, H), lambda i: (i + E // EB, 0))
    edge_h, msg = pl.pallas_call(
        _tc_edge_mlp,
        grid=(E // EB,),
        in_specs=[erow(4), src_spec, dst_spec, erow(1), erow(1), erow(1),
                  full((8, 1)), full((2, 16)), full((8, 4)), full((154, H)),
                  full((1, H)), full((H, H)), full((1, H)), full((H, H)),
                  full((1, H))],
        out_specs=[erow(H), erow(H)],
        out_shape=[jax.ShapeDtypeStruct((E, H), _f32),
                   jax.ShapeDtypeStruct((E, H), _f32)],
        compiler_params=tc_params,
    )(edge_features_t, hpair, hpair, ebi, etg, esc, etid, event_emb,
      event_params, ee_w1, b2d(ee_b1), ee_w2, b2d(ee_b2), msg_w, b2d(msg_b))

    agg = sc_scatter(msg, flat_idx)

    nhu, node_delta_pred, node_features_pred = pl.pallas_call(
        _tc_node_update,
        grid=(N // NB,),
        in_specs=[nrow(H), nrow(H), nrow(7), nrow(1), full((2 * H, H)),
                  full((1, H)), full((H, 7)), full((1, 7))],
        out_specs=[nrow(H), nrow(7), nrow(7)],
        out_shape=[jax.ShapeDtypeStruct((N, H), _f32),
                   jax.ShapeDtypeStruct((N, 7), _f32),
                   jax.ShapeDtypeStruct((N, 7), _f32)],
        compiler_params=tc_params,
    )(node_h, agg, node_features_t, nsc, nu_w, b2d(nu_b), nd_w, b2d(nd_b))

    upair = sc_gather(nhu, flat_idx)

    edge_delta_pred, edge_features_pred = pl.pallas_call(
        _tc_edge_delta,
        grid=(E // EB,),
        in_specs=[erow(H), src_spec, dst_spec, erow(4), erow(1),
                  full((3 * H, H)), full((1, H)), full((H, 4)), full((1, 4))],
        out_specs=[erow(4), erow(4)],
        out_shape=[jax.ShapeDtypeStruct((E, 4), _f32),
                   jax.ShapeDtypeStruct((E, 4), _f32)],
        compiler_params=tc_params,
    )(edge_h, upair, upair, edge_features_t, esc, ed_w1, b2d(ed_b1), ed_w2,
      b2d(ed_b2))

    return (node_delta_pred, edge_delta_pred, node_features_pred,
            edge_features_pred)


# speedup vs baseline: 3.8129x; 1.6054x over previous
import functools

import jax
import jax.numpy as jnp
from jax import lax
from jax.experimental import pallas as pl
from jax.experimental.pallas import tpu as pltpu
from jax.experimental.pallas import tpu_sc as plsc

N = 50000
E = 800000
H = 64
NB = 1000
EB = 4000

_f32 = jnp.float32


def _iota_f32(n):
    return lax.broadcasted_iota(jnp.int32, (1, n), 1).astype(_f32)


def _tc_node_mlp(nf_ref, aux_ref, etid_ref, eemb_ref,
                 eprm_ref, w1_ref, b1_ref, w2_ref, b2_ref, out_ref):
    nf = nf_ref[...]
    au = jnp.transpose(aux_ref[0])
    bidx = au[:, 0:1]
    tgt = au[:, 1:2]
    sc = au[:, 2:3]
    w1 = w1_ref[...]
    oh2 = (etid_ref[...] == _iota_f32(2)).astype(_f32)
    ge = oh2 @ eemb_ref[...]
    tw = ge @ w1[7:23] + eprm_ref[...] @ w1[23:27]
    oh8 = (bidx == _iota_f32(8)).astype(_f32)
    c1 = (nf @ w1[0:7] + oh8 @ tw + tgt * w1[27:28] + sc * w1[28:29]
          + b1_ref[...])
    h1 = jnp.maximum(c1, 0.0)
    out_ref[...] = jnp.maximum(h1 @ w2_ref[...] + b2_ref[...], 0.0)


def _tc_edge_mlp(ef_ref, hp_ref, aux_ref, etid_ref,
                 eemb_ref, eprm_ref, w1_ref, b1_ref, w2_ref, b2_ref, mw_ref,
                 mb_ref, out_ref):
    ef = ef_ref[...]
    hp = hp_ref[...]
    hs = hp[:, 0:H]
    hd = hp[:, H:2 * H]
    au = jnp.transpose(aux_ref[0])
    bidx = au[:, 0:1]
    tgt = au[:, 1:2]
    sc = au[:, 2:3]
    w1 = w1_ref[...]
    oh2 = (etid_ref[...] == _iota_f32(2)).astype(_f32)
    ge = oh2 @ eemb_ref[...]
    tw = ge @ w1[132:148] + eprm_ref[...] @ w1[148:152]
    oh8 = (bidx == _iota_f32(8)).astype(_f32)
    c1 = (ef @ w1[0:4] + hs @ w1[4:68] + hd @ w1[68:132] + oh8 @ tw
          + tgt * w1[152:153] + sc * w1[153:154] + b1_ref[...])
    e1 = jnp.maximum(c1, 0.0)
    ehv = jnp.maximum(e1 @ w2_ref[...] + b2_ref[...], 0.0)
    msg = (ehv @ mw_ref[...] + mb_ref[...]) * sc
    out_ref[...] = jnp.concatenate([ehv, msg], axis=1)


def _tc_node_update(nh_ref, ag_ref, nf_ref, aux_ref, nuw_ref, nub_ref,
                    ndw_ref, ndb_ref, nhu_ref, ndp_ref, nfp_ref):
    nh = nh_ref[...]
    ag = ag_ref[...]
    nuw = nuw_ref[...]
    nhu = jnp.maximum(nh @ nuw[0:64] + ag @ nuw[64:128] + nub_ref[...], 0.0)
    raw = nhu @ ndw_ref[...] + ndb_ref[...]
    au = jnp.transpose(aux_ref[0])
    sc = au[:, 2:3]
    nf = nf_ref[...]
    ndp = raw * sc
    nhu_ref[...] = nhu
    ndp_ref[...] = ndp
    nfp_ref[...] = jnp.where(sc > 0.5, nf + ndp, nf)


def _tc_edge_delta(em_ref, up_ref, ef_ref, aux_ref, w1_ref, b1_ref,
                   w2_ref, b2_ref, out_ref):
    w1 = w1_ref[...]
    eh = em_ref[...][:, 0:H]
    up = up_ref[...]
    t = jnp.maximum(eh @ w1[0:64] + up[:, 0:H] @ w1[64:128]
                    + up[:, H:2 * H] @ w1[128:192] + b1_ref[...], 0.0)
    raw = t @ w2_ref[...] + b2_ref[...]
    au = jnp.transpose(aux_ref[0])
    sc = au[:, 2:3]
    ef = ef_ref[...]
    edp = raw * sc
    efp = jnp.where(sc > 0.5, ef + edp, ef)
    out_ref[0] = jnp.concatenate(
        [jnp.transpose(edp), jnp.transpose(efp)], axis=0)


_SC_NW = 32
_GPW = E // _SC_NW
_GC = 1000
_GI = _GPW // _GC


def _sc_gather_body(table_hbm, idx_hbm, out_hbm, idx_v, rows_v, sem):
    wid = lax.axis_index("s") * 2 + lax.axis_index("c")
    base = wid * _GPW

    def step_src(i, carry):
        off = base + i * _GC
        pltpu.sync_copy(idx_hbm.at[pl.ds(off, _GC)], idx_v)
        pltpu.async_copy(table_hbm.at[idx_v], rows_v, sem).wait()
        pltpu.sync_copy(rows_v, out_hbm.at[pl.ds(off, _GC), pl.ds(0, H)])
        return carry

    def step_dst(i, carry):
        off = base + i * _GC
        pltpu.sync_copy(idx_hbm.at[pl.ds(E + off, _GC)], idx_v)
        pltpu.async_copy(table_hbm.at[idx_v], rows_v, sem).wait()
        pltpu.sync_copy(rows_v, out_hbm.at[pl.ds(off, _GC), pl.ds(H, H)])
        return carry

    lax.fori_loop(0, _GI, step_src, 0)
    lax.fori_loop(0, _GI, step_dst, 0)


_NHALF = N // 2
_ACC_PAD = 25008
_EPT = E // 16
_SCC = 400
_SCI = _EPT // _SCC
_WB = 250


def _sc_scatter_body(msg_hbm, idx_hbm, out_hbm, acc, mbuf, sib, dib, slb, dlb):
    c = lax.axis_index("c")
    s = lax.axis_index("s")
    nbase = c * _NHALF

    zero16 = jnp.zeros((16,), _f32)

    def zstep(r, carry):
        for k in range(4):
            mbuf[r, pl.ds(k * 16, 16)] = zero16
        return carry

    lax.fori_loop(0, _SCC, zstep, 0)
    for q in range(3):
        pltpu.sync_copy(mbuf, acc.at[pl.ds(s * 1563 + q * 400, 400)])
    pltpu.sync_copy(mbuf.at[pl.ds(0, 363)],
                    acc.at[pl.ds(s * 1563 + 1200, 363)])
    plsc.subcore_barrier()

    def step(i, carry):
        ebase = s * _EPT + i * _SCC
        pltpu.sync_copy(msg_hbm.at[pl.ds(ebase, _SCC), pl.ds(H, H)], mbuf)
        pltpu.sync_copy(idx_hbm.at[pl.ds(ebase, _SCC)], sib)
        pltpu.sync_copy(idx_hbm.at[pl.ds(E + ebase, _SCC)], dib)
        for j in range(_SCC // 16):
            sv = sib[pl.ds(j * 16, 16)] - nbase
            sv = jnp.where((sv >= 0) & (sv < _NHALF), sv, _NHALF)
            slb[j // 5, pl.ds((j % 5) * 16, 16)] = sv
            dv = dib[pl.ds(j * 16, 16)] - nbase
            dv = jnp.where((dv >= 0) & (dv < _NHALF), dv, _NHALF)
            dlb[j // 5, pl.ds((j % 5) * 16, 16)] = dv
        for q in range(5):
            pltpu.sync_copy(mbuf.at[pl.ds(q * 80, 80)], acc.at[slb.at[q]],
                            add=True)
            pltpu.sync_copy(mbuf.at[pl.ds(q * 80, 80)], acc.at[dlb.at[q]],
                            add=True)
        return carry

    lax.fori_loop(0, _SCI, step, 0)
    plsc.subcore_barrier()

    for k in range(6):
        kid = s + 16 * k
        pltpu.sync_copy(acc.at[pl.ds(kid * _WB, _WB)], mbuf.at[pl.ds(0, _WB)])
        pltpu.sync_copy(mbuf.at[pl.ds(0, _WB)],
                        out_hbm.at[pl.ds(nbase + kid * _WB, _WB)])

    @pl.when(s < 4)
    def _():
        kid = s + 96
        pltpu.sync_copy(acc.at[pl.ds(kid * _WB, _WB)], mbuf.at[pl.ds(0, _WB)])
        pltpu.sync_copy(mbuf.at[pl.ds(0, _WB)],
                        out_hbm.at[pl.ds(nbase + kid * _WB, _WB)])


@functools.cache
def _sc_kernels():
    mesh = plsc.VectorSubcoreMesh(core_axis_name="c", subcore_axis_name="s")
    sc_params = pltpu.CompilerParams(use_tc_tiling_on_sc=False)
    gather = functools.partial(
        pl.kernel,
        out_type=jax.ShapeDtypeStruct((E, 2 * H), _f32),
        mesh=mesh,
        compiler_params=sc_params,
        scratch_types=[
            pltpu.VMEM((_GC,), jnp.int32),
            pltpu.VMEM((_GC, H), _f32),
            pltpu.SemaphoreType.DMA,
        ],
    )(_sc_gather_body)
    scatter = functools.partial(
        pl.kernel,
        out_type=jax.ShapeDtypeStruct((N, H), _f32),
        mesh=mesh,
        compiler_params=sc_params,
        scratch_types=[
            pltpu.VMEM_SHARED((_ACC_PAD, H), _f32),
            pltpu.VMEM((_SCC, H), _f32),
            pltpu.VMEM((_SCC,), jnp.int32),
            pltpu.VMEM((_SCC,), jnp.int32),
            pltpu.VMEM((5, 80), jnp.int32),
            pltpu.VMEM((5, 80), jnp.int32),
        ],
    )(_sc_scatter_body)
    return gather, scatter


def kernel(node_features_t, edge_index, edge_features_t, event_type_id,
           event_params, event_node_mask, event_edge_mask,
           event_scope_node_mask, event_scope_edge_mask, node_batch_index,
           edge_batch_index, num_nodes_per_graph, num_edges_per_graph,
           event_emb, ne_w1, ne_b1, ne_w2, ne_b2, ee_w1, ee_b1, ee_w2, ee_b2,
           msg_w, msg_b, nu_w, nu_b, nd_w, nd_b, ed_w1, ed_b1, ed_w2, ed_b2):
    naux = jnp.stack([node_batch_index.astype(_f32),
                      event_node_mask.astype(_f32),
                      event_scope_node_mask.astype(_f32)])
    naux = naux.reshape(3, N // NB, NB).swapaxes(0, 1)
    eaux = jnp.stack([edge_batch_index.astype(_f32),
                      event_edge_mask.astype(_f32),
                      event_scope_edge_mask.astype(_f32)])
    eaux = eaux.reshape(3, E // EB, EB).swapaxes(0, 1)
    etid = event_type_id.astype(_f32)[:, None]
    b2d = lambda b: b[None, :]

    full = lambda shape: pl.BlockSpec(shape, lambda i: (0, 0))
    nrow = lambda w: pl.BlockSpec((NB, w), lambda i: (i, 0))
    erow = lambda w: pl.BlockSpec((EB, w), lambda i: (i, 0))
    naux_spec = pl.BlockSpec((1, 3, NB), lambda i: (i, 0, 0))
    eaux_spec = pl.BlockSpec((1, 3, EB), lambda i: (i, 0, 0))
    tc_params = pltpu.CompilerParams(dimension_semantics=("parallel",))

    node_h = pl.pallas_call(
        _tc_node_mlp,
        grid=(N // NB,),
        in_specs=[nrow(7), naux_spec, full((8, 1)),
                  full((2, 16)), full((8, 4)), full((29, H)), full((1, H)),
                  full((H, H)), full((1, H))],
        out_specs=nrow(H),
        out_shape=jax.ShapeDtypeStruct((N, H), _f32),
        compiler_params=tc_params,
    )(node_features_t, naux, etid, event_emb, event_params,
      ne_w1, b2d(ne_b1), ne_w2, b2d(ne_b2))

    sc_gather, sc_scatter = _sc_kernels()
    flat_idx = edge_index.reshape(2 * E)
    hpair = sc_gather(node_h, flat_idx)

    ehmsg = pl.pallas_call(
        _tc_edge_mlp,
        grid=(E // EB,),
        in_specs=[erow(4), erow(2 * H), eaux_spec,
                  full((8, 1)), full((2, 16)), full((8, 4)), full((154, H)),
                  full((1, H)), full((H, H)), full((1, H)), full((H, H)),
                  full((1, H))],
        out_specs=erow(2 * H),
        out_shape=jax.ShapeDtypeStruct((E, 2 * H), _f32),
        compiler_params=tc_params,
    )(edge_features_t, hpair, eaux, etid, event_emb,
      event_params, ee_w1, b2d(ee_b1), ee_w2, b2d(ee_b2), msg_w, b2d(msg_b))

    agg = sc_scatter(ehmsg, flat_idx)

    nhu, node_delta_pred, node_features_pred = pl.pallas_call(
        _tc_node_update,
        grid=(N // NB,),
        in_specs=[nrow(H), nrow(H), nrow(7), naux_spec, full((2 * H, H)),
                  full((1, H)), full((H, 7)), full((1, 7))],
        out_specs=[nrow(H), nrow(7), nrow(7)],
        out_shape=[jax.ShapeDtypeStruct((N, H), _f32),
                   jax.ShapeDtypeStruct((N, 7), _f32),
                   jax.ShapeDtypeStruct((N, 7), _f32)],
        compiler_params=tc_params,
    )(node_h, agg, node_features_t, naux, nu_w, b2d(nu_b), nd_w, b2d(nd_b))

    upair = sc_gather(nhu, flat_idx)

    edpair_t = pl.pallas_call(
        _tc_edge_delta,
        grid=(E // EB,),
        in_specs=[erow(2 * H), erow(2 * H), erow(4), eaux_spec,
                  full((3 * H, H)), full((1, H)), full((H, 4)), full((1, 4))],
        out_specs=pl.BlockSpec((1, 8, EB), lambda i: (i, 0, 0)),
        out_shape=jax.ShapeDtypeStruct((E // EB, 8, EB), _f32),
        compiler_params=tc_params,
    )(ehmsg, upair, edge_features_t, eaux, ed_w1, b2d(ed_b1), ed_w2,
      b2d(ed_b2))

    flat_t = jnp.swapaxes(edpair_t, 0, 1).reshape(8, E)
    edge_delta_pred = jnp.transpose(flat_t[0:4])
    edge_features_pred = jnp.transpose(flat_t[4:8])
    return (node_delta_pred, edge_delta_pred, node_features_pred,
            edge_features_pred)
